# baseline (speedup 1.0000x reference)
import functools

import jax
import jax.numpy as jnp
from jax import lax
from jax.experimental import pallas as pl
from jax.experimental.pallas import tpu as pltpu
from jax.experimental.pallas import tpu_sc as plsc

_BATCH = 16384
_EMB = 32
_HID = 64
_NC = 2
_NS = 16
_NW = _NC * _NS
_BPW = _BATCH // _NW
_CHUNK = 128
_NK = _BPW // _CHUNK

_ROWS = 1000000
_RPAD = 1 << 20
_DCH = 1 << 18
_NJ = _RPAD // _DCH


_BULK = (_ROWS // 128) * 128


def _detile_body(u_ref, i_ref, ut_ref, it_ref, uo_ref, io_ref, sem):
    cps = []
    for d in range(_EMB):
        for (src, tail, dst) in ((u_ref, ut_ref, uo_ref),
                                 (i_ref, it_ref, io_ref)):
            cps.append(pltpu.make_async_copy(
                src.at[d, pl.ds(0, _BULK)],
                dst.at[pl.ds(d * _RPAD, _BULK)], sem))
            cps.append(pltpu.make_async_copy(
                tail.at[d, pl.ds(0, 128)],
                dst.at[pl.ds(d * _RPAD + _BULK, 128)], sem))
    for c in cps:
        c.start()
    for c in cps:
        c.wait()


_detile = pl.pallas_call(
    _detile_body,
    in_specs=[pl.BlockSpec(memory_space=pl.ANY)] * 4,
    out_specs=[
        pl.BlockSpec(memory_space=pl.ANY),
        pl.BlockSpec(memory_space=pl.ANY),
    ],
    out_shape=[
        jax.ShapeDtypeStruct((_EMB * _RPAD,), jnp.float32),
        jax.ShapeDtypeStruct((_EMB * _RPAD,), jnp.float32),
    ],
    scratch_shapes=[pltpu.SemaphoreType.DMA],
)


def _sc_gather_body(ut, it, uidx, iidx, u_out, i_out,
                    uidx_v, iidx_v, urows_v, irows_v, sem):
    wid = lax.axis_index("s") * _NC + lax.axis_index("c")
    base = wid * _BPW
    row0 = wid * _NK
    pltpu.sync_copy(uidx.at[pl.ds(row0, _NK)], uidx_v)
    pltpu.sync_copy(iidx.at[pl.ds(row0, _NK)], iidx_v)

    def fire(d):
        cs = []
        for k in range(_NK):
            cs.append(pltpu.async_copy(
                ut.at[d].at[uidx_v.at[k]],
                urows_v.at[d, pl.ds(k * _CHUNK, _CHUNK)], sem))
            cs.append(pltpu.async_copy(
                it.at[d].at[iidx_v.at[k]],
                irows_v.at[d, pl.ds(k * _CHUNK, _CHUNK)], sem))
        return cs

    pending = fire(0)
    for d in range(1, _EMB):
        nxt = fire(d)
        for c in pending:
            c.wait()
        pending = nxt
    for c in pending:
        c.wait()

    pltpu.sync_copy(urows_v, u_out.at[:, pl.ds(base, _BPW)])
    pltpu.sync_copy(irows_v, i_out.at[:, pl.ds(base, _BPW)])


_sc_gather = functools.partial(
    pl.kernel,
    mesh=plsc.VectorSubcoreMesh(core_axis_name="c", subcore_axis_name="s"),
    out_type=[
        jax.ShapeDtypeStruct((_EMB, _BATCH), jnp.float32),
        jax.ShapeDtypeStruct((_EMB, _BATCH), jnp.float32),
    ],
    scratch_types=[
        pltpu.VMEM((_NK, _CHUNK), jnp.int32),
        pltpu.VMEM((_NK, _CHUNK), jnp.int32),
        pltpu.VMEM((_EMB, _BPW), jnp.float32),
        pltpu.VMEM((_EMB, _BPW), jnp.float32),
        pltpu.SemaphoreType.DMA,
    ],
    compiler_params=pltpu.CompilerParams(use_tc_tiling_on_sc=False),
)(_sc_gather_body)


_BN = 2048


def _mlp_body(w1u_ref, w1i_ref, b1_ref, w2_ref, b2_ref, u_ref, i_ref, o_ref):
    h = jnp.dot(w1u_ref[...], u_ref[...], preferred_element_type=jnp.float32)
    h = h + jnp.dot(w1i_ref[...], i_ref[...], preferred_element_type=jnp.float32)
    h = jnp.maximum(h + b1_ref[...], 0.0)
    o_ref[...] = jnp.dot(w2_ref[...], h, preferred_element_type=jnp.float32) + b2_ref[...]


_mlp = pl.pallas_call(
    _mlp_body,
    grid=(_BATCH // _BN,),
    in_specs=[
        pl.BlockSpec((_HID, _EMB), lambda n: (0, 0)),
        pl.BlockSpec((_HID, _EMB), lambda n: (0, 0)),
        pl.BlockSpec((_HID, 1), lambda n: (0, 0)),
        pl.BlockSpec((1, _HID), lambda n: (0, 0)),
        pl.BlockSpec((1, 1), lambda n: (0, 0)),
        pl.BlockSpec((_EMB, _BN), lambda n: (0, n)),
        pl.BlockSpec((_EMB, _BN), lambda n: (0, n)),
    ],
    out_specs=pl.BlockSpec((1, _BN), lambda n: (0, n)),
    out_shape=jax.ShapeDtypeStruct((1, _BATCH), jnp.float32),
)


def kernel(users, items, user_table, item_table, W1, b1, W2, b2):
    uidx = users.reshape(_NW * _NK, _CHUNK)
    iidx = items.reshape(_NW * _NK, _CHUNK)
    u_t0 = user_table.T
    i_t0 = item_table.T
    u_tail = jnp.pad(u_t0[:, _BULK:], ((0, 0), (0, 128 - (_ROWS - _BULK))))
    i_tail = jnp.pad(i_t0[:, _BULK:], ((0, 0), (0, 128 - (_ROWS - _BULK))))
    uflat, iflat = _detile(u_t0, i_t0, u_tail, i_tail)
    u_lin = uflat.reshape(_EMB, _RPAD)
    i_lin = iflat.reshape(_EMB, _RPAD)
    u_t, i_t = _sc_gather(u_lin, i_lin, uidx, iidx)
    w1u = W1[:, :_EMB]
    w1i = W1[:, _EMB:]
    out = _mlp(w1u, w1i, b1.reshape(_HID, 1), W2.reshape(1, _HID),
               b2.reshape(1, 1), u_t, i_t)
    return out.reshape(_BATCH)


# speedup vs baseline: 28.6892x; 28.6892x over previous
import functools

import jax
import jax.numpy as jnp
from jax import lax
from jax.experimental import pallas as pl
from jax.experimental.pallas import tpu as pltpu
from jax.experimental.pallas import tpu_sc as plsc

_BATCH = 16384
_EMB = 32
_HID = 64
_NC = 2
_NS = 16
_NW = _NC * _NS
_BPW = _BATCH // _NW
_CHUNK = 128
_NK = _BPW // _CHUNK

_ROWS = 1000000
_RPAD = 1 << 20
_DCH = 1 << 18
_NJ = _RPAD // _DCH


_DC = 1 << 15
_NJ = _RPAD // _DC
_NJIN = -(-_ROWS // _DC)


def _detile_body(u_ref, i_ref, *o_refs):
    for dd in range(8):
        o_refs[dd][...] = u_ref[dd, :]
        o_refs[8 + dd][...] = i_ref[dd, :]


_detile = pl.pallas_call(
    _detile_body,
    grid=(4, _NJIN),
    in_specs=[
        pl.BlockSpec((8, _DC), lambda db, j: (db, j)),
        pl.BlockSpec((8, _DC), lambda db, j: (db, j)),
    ],
    out_specs=[pl.BlockSpec((_DC,), lambda db, j: (db * _NJ + j,))
               for _ in range(16)],
    out_shape=[jax.ShapeDtypeStruct((4 * _RPAD,), jnp.float32)
               for _ in range(16)],
)


def _sc_gather_body(*refs):
    (uidx, iidx, u_out, i_out,
     uidx_v, iidx_v, urows_v, irows_v, sem) = refs[16:]
    ustag = refs[0:8]
    istag = refs[8:16]
    wid = lax.axis_index("s") * _NC + lax.axis_index("c")
    base = wid * _BPW
    row0 = wid * _NK
    pltpu.sync_copy(uidx.at[pl.ds(row0, _NK)], uidx_v)
    pltpu.sync_copy(iidx.at[pl.ds(row0, _NK)], iidx_v)

    def fire(d):
        db, dd = d // 8, d % 8
        cs = []
        for k in range(_NK):
            cs.append(pltpu.async_copy(
                ustag[dd].at[db].at[uidx_v.at[k]],
                urows_v.at[d, pl.ds(k * _CHUNK, _CHUNK)], sem))
            cs.append(pltpu.async_copy(
                istag[dd].at[db].at[iidx_v.at[k]],
                irows_v.at[d, pl.ds(k * _CHUNK, _CHUNK)], sem))
        return cs

    pending = fire(0)
    for d in range(1, _EMB):
        nxt = fire(d)
        for c in pending:
            c.wait()
        pending = nxt
    for c in pending:
        c.wait()

    pltpu.sync_copy(urows_v, u_out.at[:, pl.ds(base, _BPW)])
    pltpu.sync_copy(irows_v, i_out.at[:, pl.ds(base, _BPW)])


_sc_gather = functools.partial(
    pl.kernel,
    mesh=plsc.VectorSubcoreMesh(core_axis_name="c", subcore_axis_name="s"),
    out_type=[
        jax.ShapeDtypeStruct((_EMB, _BATCH), jnp.float32),
        jax.ShapeDtypeStruct((_EMB, _BATCH), jnp.float32),
    ],
    scratch_types=[
        pltpu.VMEM((_NK, _CHUNK), jnp.int32),
        pltpu.VMEM((_NK, _CHUNK), jnp.int32),
        pltpu.VMEM((_EMB, _BPW), jnp.float32),
        pltpu.VMEM((_EMB, _BPW), jnp.float32),
        pltpu.SemaphoreType.DMA,
    ],
    compiler_params=pltpu.CompilerParams(use_tc_tiling_on_sc=False),
)(_sc_gather_body)


_BN = 2048


def _mlp_body(w1u_ref, w1i_ref, b1_ref, w2_ref, b2_ref, u_ref, i_ref, o_ref):
    h = jnp.dot(w1u_ref[...], u_ref[...], preferred_element_type=jnp.float32)
    h = h + jnp.dot(w1i_ref[...], i_ref[...], preferred_element_type=jnp.float32)
    h = jnp.maximum(h + b1_ref[...], 0.0)
    o_ref[...] = jnp.dot(w2_ref[...], h, preferred_element_type=jnp.float32) + b2_ref[...]


_mlp = pl.pallas_call(
    _mlp_body,
    grid=(_BATCH // _BN,),
    in_specs=[
        pl.BlockSpec((_HID, _EMB), lambda n: (0, 0)),
        pl.BlockSpec((_HID, _EMB), lambda n: (0, 0)),
        pl.BlockSpec((_HID, 1), lambda n: (0, 0)),
        pl.BlockSpec((1, _HID), lambda n: (0, 0)),
        pl.BlockSpec((1, 1), lambda n: (0, 0)),
        pl.BlockSpec((_EMB, _BN), lambda n: (0, n)),
        pl.BlockSpec((_EMB, _BN), lambda n: (0, n)),
    ],
    out_specs=pl.BlockSpec((1, _BN), lambda n: (0, n)),
    out_shape=jax.ShapeDtypeStruct((1, _BATCH), jnp.float32),
)


def kernel(users, items, user_table, item_table, W1, b1, W2, b2):
    uidx = users.reshape(_NW * _NK, _CHUNK)
    iidx = items.reshape(_NW * _NK, _CHUNK)
    stags = _detile(user_table.T, item_table.T)
    stags = [s.reshape(4, _RPAD) for s in stags]
    u_t, i_t = _sc_gather(*stags, uidx, iidx)
    w1u = W1[:, :_EMB]
    w1i = W1[:, _EMB:]
    out = _mlp(w1u, w1i, b1.reshape(_HID, 1), W2.reshape(1, _HID),
               b2.reshape(1, 1), u_t, i_t)
    return out.reshape(_BATCH)


# speedup vs baseline: 31.1942x; 1.0873x over previous
import functools

import jax
import jax.numpy as jnp
from jax import lax
from jax.experimental import pallas as pl
from jax.experimental.pallas import tpu as pltpu
from jax.experimental.pallas import tpu_sc as plsc

_BATCH = 16384
_EMB = 32
_HID = 64
_NC = 2
_NS = 16
_NW = _NC * _NS
_BPW = _BATCH // _NW
_CHUNK = 128
_NK = _BPW // _CHUNK

_ROWS = 1000000
_RPAD = 1 << 20
_DCH = 1 << 18
_NJ = _RPAD // _DCH


_DC = 1 << 16
_NJ = _RPAD // _DC
_NJIN = -(-_ROWS // _DC)


def _detile_body(u_ref, i_ref, *o_refs):
    for dd in range(8):
        o_refs[dd][...] = u_ref[dd, :]
        o_refs[8 + dd][...] = i_ref[dd, :]


_detile = pl.pallas_call(
    _detile_body,
    grid=(4, _NJIN),
    in_specs=[
        pl.BlockSpec((8, _DC), lambda db, j: (db, j)),
        pl.BlockSpec((8, _DC), lambda db, j: (db, j)),
    ],
    out_specs=[pl.BlockSpec((_DC,), lambda db, j: (db * _NJ + j,))
               for _ in range(16)],
    out_shape=[jax.ShapeDtypeStruct((4 * _RPAD,), jnp.float32)
               for _ in range(16)],
)


def _sc_gather_body(*refs):
    (uidx, iidx, u_out, i_out,
     uidx_v, iidx_v, urows_v, irows_v, sem) = refs[16:]
    ustag = refs[0:8]
    istag = refs[8:16]
    wid = lax.axis_index("s") * _NC + lax.axis_index("c")
    base = wid * _BPW
    row0 = wid * _NK
    pltpu.sync_copy(uidx.at[pl.ds(row0, _NK)], uidx_v)
    pltpu.sync_copy(iidx.at[pl.ds(row0, _NK)], iidx_v)

    def fire(d):
        db, dd = d // 8, d % 8
        cs = []
        for k in range(_NK):
            cs.append(pltpu.async_copy(
                ustag[dd].at[db].at[uidx_v.at[k]],
                urows_v.at[d, pl.ds(k * _CHUNK, _CHUNK)], sem))
            cs.append(pltpu.async_copy(
                istag[dd].at[db].at[iidx_v.at[k]],
                irows_v.at[d, pl.ds(k * _CHUNK, _CHUNK)], sem))
        return cs

    depth = 4
    pend = [fire(d) for d in range(depth)]
    for d in range(depth, _EMB):
        nxt = fire(d)
        for c in pend.pop(0):
            c.wait()
        pend.append(nxt)
    for grp in pend:
        for c in grp:
            c.wait()

    pltpu.sync_copy(urows_v, u_out.at[:, pl.ds(base, _BPW)])
    pltpu.sync_copy(irows_v, i_out.at[:, pl.ds(base, _BPW)])


_sc_gather = functools.partial(
    pl.kernel,
    mesh=plsc.VectorSubcoreMesh(core_axis_name="c", subcore_axis_name="s"),
    out_type=[
        jax.ShapeDtypeStruct((_EMB, _BATCH), jnp.float32),
        jax.ShapeDtypeStruct((_EMB, _BATCH), jnp.float32),
    ],
    scratch_types=[
        pltpu.VMEM((_NK, _CHUNK), jnp.int32),
        pltpu.VMEM((_NK, _CHUNK), jnp.int32),
        pltpu.VMEM((_EMB, _BPW), jnp.float32),
        pltpu.VMEM((_EMB, _BPW), jnp.float32),
        pltpu.SemaphoreType.DMA,
    ],
    compiler_params=pltpu.CompilerParams(use_tc_tiling_on_sc=False),
)(_sc_gather_body)


_BN = 2048


def _mlp_body(w1u_ref, w1i_ref, b1_ref, w2_ref, b2_ref, u_ref, i_ref, o_ref):
    h = jnp.dot(w1u_ref[...], u_ref[...], preferred_element_type=jnp.float32)
    h = h + jnp.dot(w1i_ref[...], i_ref[...], preferred_element_type=jnp.float32)
    h = jnp.maximum(h + b1_ref[...], 0.0)
    o_ref[...] = jnp.dot(w2_ref[...], h, preferred_element_type=jnp.float32) + b2_ref[...]


_mlp = pl.pallas_call(
    _mlp_body,
    grid=(_BATCH // _BN,),
    in_specs=[
        pl.BlockSpec((_HID, _EMB), lambda n: (0, 0)),
        pl.BlockSpec((_HID, _EMB), lambda n: (0, 0)),
        pl.BlockSpec((_HID, 1), lambda n: (0, 0)),
        pl.BlockSpec((1, _HID), lambda n: (0, 0)),
        pl.BlockSpec((1, 1), lambda n: (0, 0)),
        pl.BlockSpec((_EMB, _BN), lambda n: (0, n)),
        pl.BlockSpec((_EMB, _BN), lambda n: (0, n)),
    ],
    out_specs=pl.BlockSpec((1, _BN), lambda n: (0, n)),
    out_shape=jax.ShapeDtypeStruct((1, _BATCH), jnp.float32),
)


def kernel(users, items, user_table, item_table, W1, b1, W2, b2):
    uidx = users.reshape(_NW * _NK, _CHUNK)
    iidx = items.reshape(_NW * _NK, _CHUNK)
    stags = _detile(user_table.T, item_table.T)
    stags = [s.reshape(4, _RPAD) for s in stags]
    u_t, i_t = _sc_gather(*stags, uidx, iidx)
    w1u = W1[:, :_EMB]
    w1i = W1[:, _EMB:]
    out = _mlp(w1u, w1i, b1.reshape(_HID, 1), W2.reshape(1, _HID),
               b2.reshape(1, 1), u_t, i_t)
    return out.reshape(_BATCH)
